# speedup vs baseline: 3.0629x; 3.0629x over previous
import functools

import jax
import jax.numpy as jnp
from jax import lax
from jax.experimental import pallas as pl
from jax.experimental.pallas import tpu as pltpu
from jax.experimental.pallas import tpu_sc as plsc

_D = 128
_NC = 2
_NS = 16
_NW = _NC * _NS
_CHUNK = 128


def _emb_body(nchunks, x_hbm, w_hbm, out_hbm, idx_v, rows_v, g0, g1):
    wid = lax.axis_index("s") * _NC + lax.axis_index("c")
    row0 = wid * (nchunks * _CHUNK)

    pltpu.sync_copy(x_hbm.at[wid], idx_v)

    sems = (g0, g1)

    def gather(c, b):
        return pltpu.make_async_copy(w_hbm.at[idx_v.at[c]], rows_v.at[b], sems[b])

    def write(c, b):
        pltpu.sync_copy(rows_v.at[b], out_hbm.at[pl.ds(row0 + c * _CHUNK, _CHUNK)])

    gather(0, 0).start()
    gather(1, 1).start()

    def body(i, carry):
        for b in range(2):
            c = 2 * i + b
            gather(c, b).wait()
            write(c, b)
            gather(c + 2, b).start()
        return carry

    lax.fori_loop(0, nchunks // 2 - 1, body, 0)

    for b in range(2):
        c = nchunks - 2 + b
        gather(c, b).wait()
        write(c, b)


def kernel(x, weight):
    S, T = x.shape
    B = S * T
    per_w = B // _NW
    nchunks = per_w // _CHUNK
    xi = x.astype(jnp.int32).reshape(_NW, nchunks, _CHUNK)

    mesh = plsc.VectorSubcoreMesh(core_axis_name="c", subcore_axis_name="s")
    k = pl.kernel(
        functools.partial(_emb_body, nchunks),
        out_type=jax.ShapeDtypeStruct((B, _D), jnp.float32),
        mesh=mesh,
        scratch_types=[
            pltpu.VMEM((nchunks, _CHUNK), jnp.int32),
            pltpu.VMEM((2, _CHUNK, _D), jnp.float32),
            pltpu.SemaphoreType.DMA,
            pltpu.SemaphoreType.DMA,
        ],
    )
    out = k(xi, weight)
    return out.reshape(S, T, _D)


# speedup vs baseline: 3.0849x; 1.0072x over previous
import functools

import jax
import jax.numpy as jnp
from jax import lax
from jax.experimental import pallas as pl
from jax.experimental.pallas import tpu as pltpu
from jax.experimental.pallas import tpu_sc as plsc

_D = 128
_NC = 2
_NS = 16
_NW = _NC * _NS
_CHUNK = 128


def _emb_body(npairs, x_hbm, w_hbm, out_hbm, idx_v, rows_v, g0, g1):
    wid = lax.axis_index("s") * _NC + lax.axis_index("c")
    pair_rows = 2 * _CHUNK
    row0 = wid * (npairs * pair_rows)

    pltpu.sync_copy(x_hbm.at[wid], idx_v)

    sems = (g0, g1)

    def gather(p, b):
        return pltpu.make_async_copy(
            w_hbm.at[idx_v.at[pl.ds(p * pair_rows, pair_rows)]],
            rows_v.at[b], sems[b])

    def write(p, b):
        pltpu.sync_copy(rows_v.at[b],
                        out_hbm.at[pl.ds(row0 + p * pair_rows, pair_rows)])

    gather(0, 0).start()
    gather(1, 1).start()

    def body(i, carry):
        for b in range(2):
            p = 2 * i + b
            gather(p, b).wait()
            write(p, b)
            gather(p + 2, b).start()
        return carry

    lax.fori_loop(0, (npairs - 3) // 2, body, 0)

    p = npairs - 3
    gather(p, 0).wait()
    write(p, 0)
    gather(p + 2, 0).start()
    gather(p + 1, 1).wait()
    write(p + 1, 1)
    gather(p + 2, 0).wait()
    write(p + 2, 0)


def kernel(x, weight):
    S, T = x.shape
    B = S * T
    per_w = B // _NW
    npairs = per_w // (2 * _CHUNK)
    xi = x.astype(jnp.int32).reshape(_NW, per_w)

    mesh = plsc.VectorSubcoreMesh(core_axis_name="c", subcore_axis_name="s")
    k = pl.kernel(
        functools.partial(_emb_body, npairs),
        out_type=jax.ShapeDtypeStruct((B, _D), jnp.float32),
        mesh=mesh,
        scratch_types=[
            pltpu.VMEM((per_w,), jnp.int32),
            pltpu.VMEM((2, 2 * _CHUNK, _D), jnp.float32),
            pltpu.SemaphoreType.DMA,
            pltpu.SemaphoreType.DMA,
        ],
    )
    out = k(xi, weight)
    return out.reshape(S, T, _D)


# speedup vs baseline: 4.7054x; 1.5253x over previous
import functools

import jax
import jax.numpy as jnp
from jax import lax
from jax.experimental import pallas as pl
from jax.experimental.pallas import tpu as pltpu
from jax.experimental.pallas import tpu_sc as plsc

_D = 128
_NC = 2
_NS = 16
_NW = _NC * _NS


def _emb_body(rows_per_w, T, x_hbm, w_hbm, out_hbm, idx_v, rows_v, g0, g1):
    wid = lax.axis_index("s") * _NC + lax.axis_index("c")
    r0 = wid * rows_per_w

    pltpu.sync_copy(x_hbm.at[pl.ds(r0, rows_per_w)], idx_v)

    sems = (g0, g1)

    def gather(r, b):
        return pltpu.make_async_copy(w_hbm.at[idx_v.at[r]], rows_v.at[b], sems[b])

    def write(r, b):
        pltpu.sync_copy(rows_v.at[b], out_hbm.at[r0 + r])

    gather(0, 0).start()
    gather(1, 1).start()

    def body(i, carry):
        for b in range(2):
            r = 2 * i + b
            gather(r, b).wait()
            write(r, b)
            gather(r + 2, b).start()
        return carry

    lax.fori_loop(0, rows_per_w // 2 - 1, body, 0)

    for b in range(2):
        r = rows_per_w - 2 + b
        gather(r, b).wait()
        write(r, b)


def kernel(x, weight):
    S, T = x.shape
    rows_per_w = S // _NW
    xi = x.astype(jnp.int32)

    mesh = plsc.VectorSubcoreMesh(core_axis_name="c", subcore_axis_name="s")
    k = pl.kernel(
        functools.partial(_emb_body, rows_per_w, T),
        out_type=jax.ShapeDtypeStruct((S, T, _D), jnp.float32),
        mesh=mesh,
        scratch_types=[
            pltpu.VMEM((rows_per_w, T), jnp.int32),
            pltpu.VMEM((2, T, _D), jnp.float32),
            pltpu.SemaphoreType.DMA,
            pltpu.SemaphoreType.DMA,
        ],
    )
    return k(xi, weight)


# speedup vs baseline: 5.4707x; 1.1626x over previous
import functools

import jax
import jax.numpy as jnp
from jax import lax
from jax.experimental import pallas as pl
from jax.experimental.pallas import tpu as pltpu
from jax.experimental.pallas import tpu_sc as plsc

_D = 128
_NC = 2
_NS = 16
_NW = _NC * _NS


def _emb_body(rows_per_w, T, group, x_hbm, w_hbm, out_hbm, idx_v, rows_v, g0, g1):
    wid = lax.axis_index("s") * _NC + lax.axis_index("c")
    r0 = wid * rows_per_w
    ngroups = rows_per_w // group

    pltpu.sync_copy(x_hbm.at[pl.ds(r0, rows_per_w)], idx_v)

    sems = (g0, g1)

    def gathers(g, b):
        return [
            pltpu.make_async_copy(
                w_hbm.at[idx_v.at[g * group + j]], rows_v.at[b, j], sems[b])
            for j in range(group)
        ]

    def fire(g, b):
        for c in gathers(g, b):
            c.start()

    def drain(g, b):
        for c in gathers(g, b):
            c.wait()

    def write(g, b):
        pltpu.sync_copy(rows_v.at[b],
                        out_hbm.at[pl.ds(r0 + g * group, group)])

    fire(0, 0)
    fire(1, 1)

    def body(i, carry):
        for b in range(2):
            g = 2 * i + b
            drain(g, b)
            write(g, b)
            fire(g + 2, b)
        return carry

    lax.fori_loop(0, ngroups // 2 - 1, body, 0)

    for b in range(2):
        g = ngroups - 2 + b
        drain(g, b)
        write(g, b)


def kernel(x, weight):
    S, T = x.shape
    rows_per_w = S // _NW
    group = 8
    xi = x.astype(jnp.int32)

    mesh = plsc.VectorSubcoreMesh(core_axis_name="c", subcore_axis_name="s")
    k = pl.kernel(
        functools.partial(_emb_body, rows_per_w, T, group),
        out_type=jax.ShapeDtypeStruct((S, T, _D), jnp.float32),
        mesh=mesh,
        scratch_types=[
            pltpu.VMEM((rows_per_w, T), jnp.int32),
            pltpu.VMEM((2, group, T, _D), jnp.float32),
            pltpu.SemaphoreType.DMA,
            pltpu.SemaphoreType.DMA,
        ],
    )
    return k(xi, weight)
